# baseline (speedup 1.0000x reference)
import functools

import jax
import jax.numpy as jnp
from jax import lax
from jax.experimental import pallas as pl
from jax.experimental.pallas import tpu as pltpu
from jax.experimental.pallas import tpu_sc as plsc

B = 16384
F = 26
C = 100
OUT_D = F * C
NW = 32
COLS_PER_W = B // NW
FPAIRS = F // 2
RROWS = 2 * C
CBLK = 256
NCL = COLS_PER_W // CBLK
GRP = CBLK // 16
L = 16

_mesh = plsc.VectorSubcoreMesh(core_axis_name="c", subcore_axis_name="s")


@functools.partial(
    pl.kernel,
    out_type=jax.ShapeDtypeStruct((OUT_D, B), jnp.float32),
    mesh=_mesh,
    scratch_types=[
        pltpu.VMEM((F, COLS_PER_W), jnp.int32),
        pltpu.VMEM((NCL, RROWS, CBLK), jnp.float32),
        pltpu.SemaphoreType.DMA((NCL,)),
        pltpu.SemaphoreType.DMA,
    ],
    compiler_params=pltpu.CompilerParams(needs_layout_passes=False),
)
def _one_hot_sc(cat_hbm, zeros_hbm, out_hbm, cat_v, bufs, sems, csem):
    cid = lax.axis_index("c")
    sid = lax.axis_index("s")
    wid = cid * (NW // 2) + sid
    cbase = wid * COLS_PER_W

    cat_cp = pltpu.make_async_copy(
        cat_hbm.at[:, pl.ds(cbase, COLS_PER_W)], cat_v, csem
    )
    cat_cp.start()
    z0_cp = pltpu.make_async_copy(zeros_hbm, bufs.at[0], sems.at[0])
    z0_cp.start()
    z1_cp = pltpu.make_async_copy(zeros_hbm.at[:], bufs.at[1], sems.at[1])
    z1_cp.start()
    cat_cp.wait()
    z0_cp.wait()
    z1_cp.wait()

    ones = jnp.ones((L,), jnp.float32)
    zeros = jnp.zeros((L,), jnp.float32)
    iota = lax.iota(jnp.int32, L)

    def scatter_pass(fp, cl, buf, vals):
        for fl in range(2):
            @pl.loop(0, GRP)
            def g_loop(g):
                v = cat_v[fp * 2 + fl, pl.ds(cl * CBLK + g * L, L)]
                v = jnp.minimum(jnp.maximum(v, 0), C - 1)
                plsc.store_scatter(buf, [fl * C + v, iota + g * L], vals)

    def out_slice(fp, cl):
        return out_hbm.at[
            pl.ds(fp * RROWS, RROWS), pl.ds(cbase + cl * CBLK, CBLK)
        ]

    @pl.loop(0, FPAIRS * NCL)
    def t_loop(t):
        fp = t // NCL
        cl = lax.rem(t, NCL)
        buf = bufs.at[cl]
        sem = sems.at[cl]

        @pl.when(t >= NCL)
        def _clear():
            pltpu.make_async_copy(buf, out_slice(fp - 1, cl), sem).wait()
            scatter_pass(fp - 1, cl, buf, zeros)

        scatter_pass(fp, cl, buf, ones)
        pltpu.make_async_copy(buf, out_slice(fp, cl), sem).start()

    for cl in range(NCL):
        pltpu.make_async_copy(
            bufs.at[cl], out_slice(FPAIRS - 1, cl), sems.at[cl]
        ).wait()


@jax.jit
def _run(cat_tensor):
    cat_t = cat_tensor.T.astype(jnp.int32)
    zeros = jnp.zeros((RROWS, CBLK), jnp.float32)
    return _one_hot_sc(cat_t, zeros).T


def kernel(cat_tensor):
    if cat_tensor.ndim == 1:
        cat_tensor = cat_tensor[None, :]
    return _run(cat_tensor)


# speedup vs baseline: 1.0919x; 1.0919x over previous
import functools

import jax
import jax.numpy as jnp
from jax import lax
from jax.experimental import pallas as pl
from jax.experimental.pallas import tpu as pltpu
from jax.experimental.pallas import tpu_sc as plsc

B = 16384
F = 26
C = 100
OUT_D = F * C
NW = 32
COLS_PER_W = B // NW
FPAIRS = F // 2
RROWS = 2 * C
CBLK = 256
NCL = COLS_PER_W // CBLK
GRP = CBLK // 16
L = 16

_mesh = plsc.VectorSubcoreMesh(core_axis_name="c", subcore_axis_name="s")


@functools.partial(
    pl.kernel,
    out_type=jax.ShapeDtypeStruct((OUT_D, B), jnp.float32),
    mesh=_mesh,
    scratch_types=[
        pltpu.VMEM((F, COLS_PER_W), jnp.int32),
        pltpu.VMEM((RROWS, CBLK), jnp.float32),
        pltpu.VMEM((RROWS, CBLK), jnp.float32),
        pltpu.SemaphoreType.DMA,
        pltpu.SemaphoreType.DMA,
    ],
    compiler_params=pltpu.CompilerParams(needs_layout_passes=False),
)
def _one_hot_sc(cat_hbm, zeros_hbm, out_hbm, cat_v, buf0, buf1, sem0, sem1):
    cid = lax.axis_index("c")
    sid = lax.axis_index("s")
    wid = cid * (NW // 2) + sid
    cbase = wid * COLS_PER_W
    bufs = (buf0, buf1)
    sems = (sem0, sem1)

    cat_cp = pltpu.make_async_copy(
        cat_hbm.at[:, pl.ds(cbase, COLS_PER_W)], cat_v, sem0
    )
    cat_cp.start()
    z0_cp = pltpu.make_async_copy(zeros_hbm, buf0, sem1)
    z0_cp.start()
    z1_cp = pltpu.make_async_copy(zeros_hbm.at[:], buf1, sem1)
    z1_cp.start()
    cat_cp.wait()
    z0_cp.wait()
    z1_cp.wait()

    ones = jnp.ones((L,), jnp.float32)
    zeros = jnp.zeros((L,), jnp.float32)
    iota = lax.iota(jnp.int32, L)

    def scatter_pass(fp, cl, buf, vals):
        for fl in range(2):
            @pl.loop(0, GRP)
            def g_loop(g):
                v = cat_v[fp * 2 + fl, pl.ds(cl * CBLK + g * L, L)]
                v = jnp.minimum(jnp.maximum(v, 0), C - 1)
                plsc.store_scatter(buf, [fl * C + v, iota + g * L], vals)

    def out_slice(fp, cl):
        return out_hbm.at[
            pl.ds(fp * RROWS, RROWS), pl.ds(cbase + cl * CBLK, CBLK)
        ]

    @pl.loop(0, FPAIRS)
    def fp_loop(fp):
        for cl in range(NCL):
            @pl.when(fp > 0)
            def _clear():
                pltpu.make_async_copy(
                    bufs[cl], out_slice(fp - 1, cl), sems[cl]
                ).wait()
                scatter_pass(fp - 1, cl, bufs[cl], zeros)

            scatter_pass(fp, cl, bufs[cl], ones)
            pltpu.make_async_copy(bufs[cl], out_slice(fp, cl), sems[cl]).start()

    for cl in range(NCL):
        pltpu.make_async_copy(bufs[cl], out_slice(0, cl), sems[cl]).wait()


@jax.jit
def _run(cat_tensor):
    cat_t = cat_tensor.T.astype(jnp.int32)
    zeros = jnp.zeros((RROWS, CBLK), jnp.float32)
    return _one_hot_sc(cat_t, zeros).T


def kernel(cat_tensor):
    if cat_tensor.ndim == 1:
        cat_tensor = cat_tensor[None, :]
    return _run(cat_tensor)
